# baseline (speedup 1.0000x reference)
import jax
import jax.numpy as jnp
from jax.experimental import pallas as pl

K = 129
D_MODEL = 1024
NUM_HEADS = 16
SCALE = 32.0


def _body(wq_ref, wk_ref, wv_ref, oq_ref, ok_ref, ov_ref):
    oq_ref[0, 0] = wq_ref[...] * SCALE
    ok_ref[0, 0] = wk_ref[...] * SCALE
    ov_ref[0, 0] = wv_ref[...] * SCALE


def kernel(Wq, Wk, Wv):
    in_spec = pl.BlockSpec((K, D_MODEL), lambda h: (0, 0))
    out_spec = pl.BlockSpec((1, 1, K, D_MODEL), lambda h: (0, h, 0, 0))
    out_shape = jax.ShapeDtypeStruct((1, NUM_HEADS, K, D_MODEL), jnp.float32)
    return pl.pallas_call(
        _body,
        grid=(NUM_HEADS,),
        in_specs=[in_spec, in_spec, in_spec],
        out_specs=[out_spec, out_spec, out_spec],
        out_shape=[out_shape, out_shape, out_shape],
    )(Wq, Wk, Wv)


# speedup vs baseline: 1.0413x; 1.0413x over previous
import jax
import jax.numpy as jnp
from jax.experimental import pallas as pl
from jax.experimental.pallas import tpu as pltpu

K = 129
D_MODEL = 1024
NUM_HEADS = 16
SCALE = 32.0


HEADS_PER_STEP = 4


def _body(wq_ref, wk_ref, wv_ref, oq_ref, ok_ref, ov_ref):
    for o_ref, w_ref in ((oq_ref, wq_ref), (ok_ref, wk_ref), (ov_ref, wv_ref)):
        w = w_ref[...] * SCALE
        o_ref[0] = jnp.broadcast_to(w[None], (HEADS_PER_STEP, K, D_MODEL))


def kernel(Wq, Wk, Wv):
    in_spec = pl.BlockSpec((K, D_MODEL), lambda h: (0, 0))
    out_spec = pl.BlockSpec(
        (1, HEADS_PER_STEP, K, D_MODEL), lambda h: (0, h, 0, 0)
    )
    out_shape = jax.ShapeDtypeStruct((1, NUM_HEADS, K, D_MODEL), jnp.float32)
    return pl.pallas_call(
        _body,
        grid=(NUM_HEADS // HEADS_PER_STEP,),
        in_specs=[in_spec, in_spec, in_spec],
        out_specs=[out_spec, out_spec, out_spec],
        out_shape=[out_shape, out_shape, out_shape],
        compiler_params=pltpu.CompilerParams(
            dimension_semantics=("parallel",)
        ),
    )(Wq, Wk, Wv)


# speedup vs baseline: 1.0487x; 1.0071x over previous
import jax
import jax.numpy as jnp
from jax.experimental import pallas as pl
from jax.experimental.pallas import tpu as pltpu

K = 129
D_MODEL = 1024
NUM_HEADS = 16
SCALE = 32.0


def _body(wq_hbm, wk_hbm, wv_hbm, oq_hbm, ok_hbm, ov_hbm,
          vq, vk, vv, sem_in, sem_out):
    pairs = ((wq_hbm, vq), (wk_hbm, vk), (wv_hbm, vv))
    in_copies = [pltpu.make_async_copy(src, dst, sem_in) for src, dst in pairs]
    for c in in_copies:
        c.start()
    for c in in_copies:
        c.wait()

    vq[...] = vq[...] * SCALE
    vk[...] = vk[...] * SCALE
    vv[...] = vv[...] * SCALE

    out_copies = []
    for v, o in ((vq, oq_hbm), (vk, ok_hbm), (vv, ov_hbm)):
        for h in range(NUM_HEADS):
            out_copies.append(
                pltpu.make_async_copy(v, o.at[0, h], sem_out)
            )
    for c in out_copies:
        c.start()
    for c in out_copies:
        c.wait()


def kernel(Wq, Wk, Wv):
    any_spec = pl.BlockSpec(memory_space=pltpu.MemorySpace.HBM)
    out_shape = jax.ShapeDtypeStruct((1, NUM_HEADS, K, D_MODEL), jnp.float32)
    return pl.pallas_call(
        _body,
        in_specs=[any_spec, any_spec, any_spec],
        out_specs=[any_spec, any_spec, any_spec],
        out_shape=[out_shape, out_shape, out_shape],
        scratch_shapes=[
            pltpu.VMEM((K, D_MODEL), jnp.float32),
            pltpu.VMEM((K, D_MODEL), jnp.float32),
            pltpu.VMEM((K, D_MODEL), jnp.float32),
            pltpu.SemaphoreType.DMA,
            pltpu.SemaphoreType.DMA,
        ],
    )(Wq, Wk, Wv)
